# baseline (speedup 1.0000x reference)
import functools

import jax
import jax.numpy as jnp
from jax import lax
from jax.experimental import pallas as pl
from jax.experimental.pallas import tpu as pltpu
from jax.experimental.pallas import tpu_sc as plsc

_NUM_CLASSES = 1000
_HIDDEN = 1152
_BATCH = 16384
_ROWS = _NUM_CLASSES + 1

_NC = 2
_NS = 16
_LANES = 16
_HALF = _BATCH // _NC
_COLS = _HIDDEN // _NS
_G = 128
_NBLK = _HALF // _G
_GPB = _G // _LANES
_PAD = _COLS + 1

_mesh = plsc.VectorSubcoreMesh(core_axis_name="c", subcore_axis_name="s")


@functools.partial(
    pl.kernel,
    mesh=_mesh,
    out_type=jax.ShapeDtypeStruct((_BATCH, _HIDDEN), jnp.float32),
    scratch_types=[
        pltpu.VMEM((_ROWS, _PAD), jnp.float32),
        pltpu.VMEM((_HALF,), jnp.int32),
        pltpu.VMEM((_HALF,), jnp.int32),
        pltpu.VMEM((_G, _PAD), jnp.float32),
    ],
    compiler_params=pltpu.CompilerParams(
        use_tc_tiling_on_sc=False, needs_layout_passes=False),
)
def _embed(labels_hbm, force_hbm, table_hbm, out_hbm,
           tabcol, idx_v, frc_v, stage):
    cid = lax.axis_index("c")
    sid = lax.axis_index("s")
    half = pl.multiple_of(cid * _HALF, _HALF)
    col0 = pl.multiple_of(sid * _COLS, _COLS)

    pltpu.sync_copy(table_hbm.at[:, pl.ds(col0, _COLS)],
                    tabcol.at[:, pl.ds(0, _COLS)])
    pltpu.sync_copy(labels_hbm.at[pl.ds(half, _HALF)], idx_v)
    pltpu.sync_copy(force_hbm.at[pl.ds(half, _HALF)], frc_v)

    @pl.loop(0, _HALF // _LANES)
    def _(i):
        sl = pl.ds(pl.multiple_of(i * _LANES, _LANES), _LANES)
        idx_v[sl] = jnp.where(frc_v[sl] == 1, _NUM_CLASSES, idx_v[sl])

    iota = lax.iota(jnp.int32, _LANES)

    @pl.loop(0, _NBLK)
    def _(blk):
        row0 = pl.multiple_of(blk * _G, _G)
        for k in range(_GPB):
            idx16 = idx_v[pl.ds(row0 + k * _LANES, _LANES)]
            rows16 = iota + (k * _LANES)
            for j in range(_COLS):
                jv = jnp.full((_LANES,), j, jnp.int32)
                v = plsc.load_gather(tabcol, [idx16, jv])
                plsc.store_scatter(stage, [rows16, jv], v)
        pltpu.sync_copy(
            stage.at[:, pl.ds(0, _COLS)],
            out_hbm.at[pl.ds(half + row0, _G), pl.ds(col0, _COLS)])


def kernel(labels, train, force_drop_ids, embedding_table):
    del train
    return _embed(labels.astype(jnp.int32),
                  force_drop_ids.astype(jnp.int32),
                  embedding_table)


# speedup vs baseline: 1.9741x; 1.9741x over previous
import functools

import jax
import jax.numpy as jnp
from jax import lax
from jax.experimental import pallas as pl
from jax.experimental.pallas import tpu as pltpu
from jax.experimental.pallas import tpu_sc as plsc

_NUM_CLASSES = 1000
_HIDDEN = 1152
_BATCH = 16384
_ROWS = _NUM_CLASSES + 1

_NC = 2
_NS = 16
_LANES = 16
_HALF = _BATCH // _NC
_COLS = _HIDDEN // _NS
_G = 128
_NBLK = _HALF // _G
_GPB = _G // _LANES

_mesh = plsc.VectorSubcoreMesh(core_axis_name="c", subcore_axis_name="s")


@functools.partial(
    pl.kernel,
    mesh=_mesh,
    out_type=jax.ShapeDtypeStruct((_BATCH, _HIDDEN), jnp.float32),
    scratch_types=[
        pltpu.VMEM((_ROWS, _COLS), jnp.float32),
        pltpu.VMEM((_HALF,), jnp.int32),
        pltpu.VMEM((_HALF,), jnp.int32),
        pltpu.VMEM((_G, _COLS), jnp.float32),
    ],
    compiler_params=pltpu.CompilerParams(
        use_tc_tiling_on_sc=False, needs_layout_passes=False),
)
def _embed(labels_hbm, force_hbm, table_hbm, out_hbm,
           tabcol, idx_v, frc_v, stage):
    cid = lax.axis_index("c")
    sid = lax.axis_index("s")
    half = pl.multiple_of(cid * _HALF, _HALF)
    col0 = pl.multiple_of(sid * _COLS, _COLS)

    pltpu.sync_copy(table_hbm.at[:, pl.ds(col0, _COLS)], tabcol)
    pltpu.sync_copy(labels_hbm.at[pl.ds(half, _HALF)], idx_v)
    pltpu.sync_copy(force_hbm.at[pl.ds(half, _HALF)], frc_v)

    @pl.loop(0, _HALF // _LANES)
    def _(i):
        sl = pl.ds(pl.multiple_of(i * _LANES, _LANES), _LANES)
        idx_v[sl] = jnp.where(frc_v[sl] == 1, _NUM_CLASSES, idx_v[sl])

    iota = lax.iota(jnp.int32, _LANES)

    @pl.loop(0, _NBLK)
    def _(blk):
        row0 = pl.multiple_of(blk * _G, _G)
        for k in range(_GPB):
            idx16 = idx_v[pl.ds(row0 + k * _LANES, _LANES)]
            rows16 = iota + (k * _LANES)
            for jb in range(0, _COLS, 16):
                n = min(16, _COLS - jb)
                vals = [plsc.load_gather(
                            tabcol,
                            [idx16, jnp.full((_LANES,), jb + t, jnp.int32)])
                        for t in range(n)]
                for t in range(n):
                    plsc.store_scatter(
                        stage,
                        [rows16, jnp.full((_LANES,), jb + t, jnp.int32)],
                        vals[t])
        pltpu.sync_copy(
            stage, out_hbm.at[pl.ds(half + row0, _G), pl.ds(col0, _COLS)])


def kernel(labels, train, force_drop_ids, embedding_table):
    del train
    return _embed(labels.astype(jnp.int32),
                  force_drop_ids.astype(jnp.int32),
                  embedding_table)


# speedup vs baseline: 3.7984x; 1.9241x over previous
import functools

import jax
import jax.numpy as jnp
from jax import lax
from jax.experimental import pallas as pl
from jax.experimental.pallas import tpu as pltpu
from jax.experimental.pallas import tpu_sc as plsc

_NUM_CLASSES = 1000
_HIDDEN = 1152
_BATCH = 16384
_ROWS = _NUM_CLASSES + 1

_NC = 2
_NS = 16
_NW = _NC * _NS
_BPW = _BATCH // _NW
_GRP = 16
_NGRP = _BPW // _GRP

_STG = 64
_STG_LAST = _ROWS - 15 * _STG

_mesh = plsc.VectorSubcoreMesh(core_axis_name="c", subcore_axis_name="s")


@functools.partial(
    pl.kernel,
    mesh=_mesh,
    out_type=jax.ShapeDtypeStruct((_BATCH, _HIDDEN), jnp.float32),
    scratch_types=[
        pltpu.VMEM_SHARED((_ROWS, _HIDDEN), jnp.float32),
        pltpu.VMEM((2, _GRP, _HIDDEN), jnp.float32),
        pltpu.VMEM((_BPW,), jnp.int32),
        pltpu.VMEM((_BPW,), jnp.int32),
        pltpu.SemaphoreType.DMA,
        pltpu.SemaphoreType.DMA,
    ],
    compiler_params=pltpu.CompilerParams(
        use_tc_tiling_on_sc=False, needs_layout_passes=False),
)
def _embed(labels_hbm, force_hbm, table_hbm, out_hbm,
           table_sp, rowbuf, idx_v, frc_v, rsem, osem):
    cid = lax.axis_index("c")
    sid = lax.axis_index("s")
    wid = sid * _NC + cid
    base = pl.multiple_of(wid * _BPW, _BPW)

    @pl.when(sid < 15)
    def _():
        off = pl.multiple_of(sid * _STG, _STG)
        pltpu.sync_copy(table_hbm.at[pl.ds(off, _STG)],
                        table_sp.at[pl.ds(off, _STG)])

    @pl.when(sid == 15)
    def _():
        pltpu.sync_copy(table_hbm.at[pl.ds(15 * _STG, _STG_LAST)],
                        table_sp.at[pl.ds(15 * _STG, _STG_LAST)])

    pltpu.sync_copy(labels_hbm.at[pl.ds(base, _BPW)], idx_v)
    pltpu.sync_copy(force_hbm.at[pl.ds(base, _BPW)], frc_v)

    for i in range(_BPW // 16):
        sl = pl.ds(i * 16, 16)
        idx_v[sl] = jnp.where(frc_v[sl] == 1, _NUM_CLASSES, idx_v[sl])

    plsc.subcore_barrier()

    def gather_group(g, b):
        idx16 = idx_v[pl.ds(pl.multiple_of(g * _GRP, _GRP), _GRP)]
        copies = []
        for r in range(_GRP):
            i = idx16[r]
            copies.append(
                pltpu.async_copy(table_sp.at[i], rowbuf.at[b, r], rsem))
        for cp in copies:
            cp.wait()

    @pl.loop(0, _NGRP, step=2)
    def _(g0):
        for b in range(2):
            g = g0 + b

            @pl.when(g >= 2)
            def _():
                pltpu.make_async_copy(
                    out_hbm.at[pl.ds(0, _GRP)], rowbuf.at[b], osem).wait()

            gather_group(g, b)
            pltpu.async_copy(
                rowbuf.at[b], out_hbm.at[pl.ds(base + g * _GRP, _GRP)], osem)

    for b in range(2):
        pltpu.make_async_copy(
            out_hbm.at[pl.ds(0, _GRP)], rowbuf.at[b], osem).wait()


def kernel(labels, train, force_drop_ids, embedding_table):
    del train
    return _embed(labels.astype(jnp.int32),
                  force_drop_ids.astype(jnp.int32),
                  embedding_table)
